# initial kernel scaffold (unmeasured)
import jax
import jax.numpy as jnp
from jax import lax
from jax.experimental import pallas as pl
from jax.experimental.pallas import tpu as pltpu


def kernel(
    x,
):
    def body(*refs):
        pass

    out_shape = jax.ShapeDtypeStruct(..., jnp.float32)
    return pl.pallas_call(body, out_shape=out_shape)(...)


# baseline (device time: 236931 ns/iter reference)
import jax
import jax.numpy as jnp
from jax import lax
from jax.experimental import pallas as pl
from jax.experimental.pallas import tpu as pltpu


def kernel(x):
    m_shard, n = x.shape
    m_half = m_shard // 2
    m_chunk = 1024
    n_chunks = m_shard // m_chunk

    def body(x_ref, out_ref, stage, copy_sems, send_sems, recv_sems):
        my_x = lax.axis_index("x")
        my_y = lax.axis_index("y")

        barrier_sem = pltpu.get_barrier_semaphore()
        pl.semaphore_signal(
            barrier_sem, inc=1,
            device_id=(my_x, 1 - my_y), device_id_type=pl.DeviceIdType.MESH,
        )
        pl.semaphore_signal(
            barrier_sem, inc=1,
            device_id=(1 - my_x, my_y), device_id_type=pl.DeviceIdType.MESH,
        )

        for i in range(n_chunks):
            slot = i % 2
            dma = pltpu.make_async_copy(
                x_ref.at[pl.ds(i * m_chunk, m_chunk), :],
                stage.at[slot],
                copy_sems.at[slot],
            )
            dma.start()
            dma.wait()
            out_ref[pl.ds(my_y * m_shard + i * m_chunk, m_chunk), :] = stage[
                slot
            ].astype(out_ref.dtype)

        pl.semaphore_wait(barrier_sem, 2)

        off1 = my_y * m_shard + my_x * m_half
        rdma1 = pltpu.make_async_remote_copy(
            src_ref=out_ref.at[pl.ds(off1, m_half), :],
            dst_ref=out_ref.at[pl.ds(off1, m_half), :],
            send_sem=send_sems.at[0],
            recv_sem=recv_sems.at[0],
            device_id=(my_x, 1 - my_y),
            device_id_type=pl.DeviceIdType.MESH,
        )
        rdma1.start()
        rdma1.wait()

        off2 = (1 - my_y) * m_shard + my_x * m_half
        rdma2 = pltpu.make_async_remote_copy(
            src_ref=out_ref.at[pl.ds(off2, m_half), :],
            dst_ref=out_ref.at[pl.ds(off2, m_half), :],
            send_sem=send_sems.at[1],
            recv_sem=recv_sems.at[1],
            device_id=(1 - my_x, my_y),
            device_id_type=pl.DeviceIdType.MESH,
        )
        rdma2.start()
        rdma2.wait()

    return pl.pallas_call(
        body,
        out_shape=jax.ShapeDtypeStruct((2 * m_shard, n), jnp.bfloat16),
        in_specs=[pl.BlockSpec(memory_space=pltpu.MemorySpace.HBM)],
        out_specs=pl.BlockSpec(memory_space=pltpu.VMEM),
        scratch_shapes=[
            pltpu.VMEM((2, m_chunk, n), jnp.float32),
            pltpu.SemaphoreType.DMA((2,)),
            pltpu.SemaphoreType.DMA((2,)),
            pltpu.SemaphoreType.DMA((2,)),
        ],
        compiler_params=pltpu.CompilerParams(
            collective_id=0,
            vmem_limit_bytes=60 * 1024 * 1024,
        ),
    )(x)


# device time: 141018 ns/iter; 1.6801x vs baseline; 1.6801x over previous
import jax
import jax.numpy as jnp
from jax import lax
from jax.experimental import pallas as pl
from jax.experimental.pallas import tpu as pltpu


def kernel(x):
    m_shard, n = x.shape
    m_half = m_shard // 2
    rc = 512
    n_comm = m_half // rc
    n_load = m_shard // rc
    n_slots = 4

    def body(x_ref, out_ref, stage, load_sems, s1_sems, r1_sems, s2_sems,
             r2_sems):
        my_x = lax.axis_index("x")
        my_y = lax.axis_index("y")

        barrier_sem = pltpu.get_barrier_semaphore()
        pl.semaphore_signal(
            barrier_sem, inc=1,
            device_id=(my_x, 1 - my_y), device_id_type=pl.DeviceIdType.MESH,
        )
        pl.semaphore_signal(
            barrier_sem, inc=1,
            device_id=(1 - my_x, my_y), device_id_type=pl.DeviceIdType.MESH,
        )

        off1 = my_y * m_shard + my_x * m_half
        off2 = (1 - my_y) * m_shard + my_x * m_half

        def local_off(j):
            if j < n_comm:
                return my_x * m_half + j * rc
            return (1 - my_x) * m_half + (j - n_comm) * rc

        def start_load(j):
            dma = pltpu.make_async_copy(
                x_ref.at[pl.ds(local_off(j), rc), :],
                stage.at[j % n_slots],
                load_sems.at[j % n_slots],
            )
            dma.start()
            return dma

        loads = {j: start_load(j) for j in range(n_slots)}

        pl.semaphore_wait(barrier_sem, 2)

        p1 = []
        for j in range(n_load):
            loads[j].wait()
            out_ref[pl.ds(my_y * m_shard + local_off(j), rc), :] = stage[
                j % n_slots
            ].astype(out_ref.dtype)
            if j + n_slots < n_load:
                loads[j + n_slots] = start_load(j + n_slots)
            if j < n_comm:
                rdma = pltpu.make_async_remote_copy(
                    src_ref=out_ref.at[pl.ds(off1 + j * rc, rc), :],
                    dst_ref=out_ref.at[pl.ds(off1 + j * rc, rc), :],
                    send_sem=s1_sems.at[j],
                    recv_sem=r1_sems.at[j],
                    device_id=(my_x, 1 - my_y),
                    device_id_type=pl.DeviceIdType.MESH,
                )
                rdma.start()
                p1.append(rdma)

        p2 = []
        for j in range(n_comm):
            p1[j].wait_recv()
            rdma = pltpu.make_async_remote_copy(
                src_ref=out_ref.at[pl.ds(off2 + j * rc, rc), :],
                dst_ref=out_ref.at[pl.ds(off2 + j * rc, rc), :],
                send_sem=s2_sems.at[j],
                recv_sem=r2_sems.at[j],
                device_id=(1 - my_x, my_y),
                device_id_type=pl.DeviceIdType.MESH,
            )
            rdma.start()
            p2.append(rdma)

        for j in range(n_comm):
            p2[j].wait_recv()
        for j in range(n_comm):
            p1[j].wait_send()
            p2[j].wait_send()

    return pl.pallas_call(
        body,
        out_shape=jax.ShapeDtypeStruct((2 * m_shard, n), jnp.bfloat16),
        in_specs=[pl.BlockSpec(memory_space=pltpu.MemorySpace.HBM)],
        out_specs=pl.BlockSpec(memory_space=pltpu.VMEM),
        scratch_shapes=[
            pltpu.VMEM((n_slots, rc, n), jnp.float32),
            pltpu.SemaphoreType.DMA((n_slots,)),
            pltpu.SemaphoreType.DMA((n_comm,)),
            pltpu.SemaphoreType.DMA((n_comm,)),
            pltpu.SemaphoreType.DMA((n_comm,)),
            pltpu.SemaphoreType.DMA((n_comm,)),
        ],
        compiler_params=pltpu.CompilerParams(
            collective_id=0,
            vmem_limit_bytes=60 * 1024 * 1024,
        ),
    )(x)
